# baseline (speedup 1.0000x reference)
import functools

import jax
import jax.numpy as jnp
from jax import lax
from jax.experimental import pallas as pl
from jax.experimental.pallas import tpu as pltpu
from jax.experimental.pallas import tpu_sc as plsc

HIDDEN = 128
NUM_RBF = 50
N_NODES = 10000
N_EDGES = 320000
CUTOFF_UPPER = 5.0


ROW_BLK = 1000


def _bf16_pair_pack(a, b):
    lo = lax.bitcast_convert_type(a.astype(jnp.bfloat16), jnp.uint16).astype(jnp.uint32)
    hi = lax.bitcast_convert_type(b.astype(jnp.bfloat16), jnp.uint16).astype(jnp.uint32)
    return lax.bitcast_convert_type(lo | (hi << 16), jnp.int32)


def _qkv_body(x_ref, wq_ref, bq_ref, wk_ref, bk_ref, wv_ref, bv_ref,
              ln_g_ref, ln_b_ref, q_ref, kv_ref):
    x = x_ref[...]
    mu = jnp.mean(x, axis=-1, keepdims=True)
    var = jnp.mean((x - mu) ** 2, axis=-1, keepdims=True)
    xn = (x - mu) / jnp.sqrt(var + 1e-5) * ln_g_ref[...] + ln_b_ref[...]
    q_ref[...] = jnp.dot(xn, wq_ref[...], preferred_element_type=jnp.float32) + bq_ref[...]
    k = jnp.dot(xn, wk_ref[...], preferred_element_type=jnp.float32) + bk_ref[...]
    v = jnp.dot(xn, wv_ref[...], preferred_element_type=jnp.float32) + bv_ref[...]
    kv_ref[...] = _bf16_pair_pack(k, v)


def _qkv(x, WqT, bq, WkT, bk, WvT, bv, ln_g, ln_b):
    n = x.shape[0]
    grid = n // ROW_BLK
    row_spec = pl.BlockSpec((ROW_BLK, HIDDEN), lambda i: (i, 0))
    full_spec = pl.BlockSpec((HIDDEN, HIDDEN), lambda i: (0, 0))
    vec_spec = pl.BlockSpec((1, HIDDEN), lambda i: (0, 0))
    out_sd = jax.ShapeDtypeStruct((n, HIDDEN), jnp.float32)
    out_sd_i = jax.ShapeDtypeStruct((n, HIDDEN), jnp.int32)
    return pl.pallas_call(
        _qkv_body,
        grid=(grid,),
        in_specs=[row_spec, full_spec, vec_spec, full_spec, vec_spec,
                  full_spec, vec_spec, vec_spec, vec_spec],
        out_specs=[row_spec, row_spec],
        out_shape=[out_sd, out_sd_i],
    )(x, WqT, bq.reshape(1, -1), WkT, bk.reshape(1, -1), WvT, bv.reshape(1, -1),
      ln_g.reshape(1, -1), ln_b.reshape(1, -1))


EDGE_BLK = 2560


def _silu(x):
    return x * jax.nn.sigmoid(x)


def _dkdv_body(attr_ref, w_ref, wdk_ref, bdk_ref, wdv_ref, bdv_ref, dk_ref):
    attr = attr_ref[...]
    dk = _silu(jnp.dot(attr, wdk_ref[...], preferred_element_type=jnp.float32) + bdk_ref[...])
    dv = _silu(jnp.dot(attr, wdv_ref[...], preferred_element_type=jnp.float32) + bdv_ref[...])
    r = w_ref[...]
    cut = 0.5 * (jnp.cos(r * (jnp.pi / CUTOFF_UPPER)) + 1.0)
    cut = cut * (r < CUTOFF_UPPER).astype(jnp.float32)
    dk_ref[...] = _bf16_pair_pack(dk, dv * cut[:, :, None].reshape(-1, 1))


def _dkdv(edge_attr, edge_weight, WdkT, bdk, WdvT, bdv):
    e = edge_attr.shape[0]
    grid = e // EDGE_BLK
    attr_spec = pl.BlockSpec((EDGE_BLK, NUM_RBF), lambda i: (i, 0))
    w_spec = pl.BlockSpec((1, EDGE_BLK), lambda i: (0, i))
    wm_spec = pl.BlockSpec((NUM_RBF, HIDDEN), lambda i: (0, 0))
    vec_spec = pl.BlockSpec((1, HIDDEN), lambda i: (0, 0))
    out_spec = pl.BlockSpec((EDGE_BLK, HIDDEN), lambda i: (i, 0))
    out_sd = jax.ShapeDtypeStruct((e, HIDDEN), jnp.int32)
    return pl.pallas_call(
        _dkdv_body,
        grid=(grid,),
        in_specs=[attr_spec, w_spec, wm_spec, vec_spec, wm_spec, vec_spec],
        out_specs=out_spec,
        out_shape=out_sd,
    )(edge_attr, edge_weight.reshape(1, -1), WdkT, bdk.reshape(1, -1),
      WdvT, bdv.reshape(1, -1))


def _final_body(x_ref, p0_ref, p1_ref, wo_ref, bo_ref, out_ref):
    s = p0_ref[...] + p1_ref[...]
    out_ref[...] = x_ref[...] + jnp.dot(s, wo_ref[...], preferred_element_type=jnp.float32) + bo_ref[...]


def _final(x, p0, p1, WoT, bo):
    n = x.shape[0]
    grid = n // ROW_BLK
    row_spec = pl.BlockSpec((ROW_BLK, HIDDEN), lambda i: (i, 0))
    full_spec = pl.BlockSpec((HIDDEN, HIDDEN), lambda i: (0, 0))
    vec_spec = pl.BlockSpec((1, HIDDEN), lambda i: (0, 0))
    return pl.pallas_call(
        _final_body,
        grid=(grid,),
        in_specs=[row_spec, row_spec, row_spec, full_spec, vec_spec],
        out_specs=row_spec,
        out_shape=jax.ShapeDtypeStruct((n, HIDDEN), jnp.float32),
    )(x, p0, p1, WoT, bo.reshape(1, -1))


_NC = 2
_NS = 16
_LANES = 16
_NW = _NC * _NS
_EPW = N_EDGES // _NW
_CHUNK = 48
_NFULL = _EPW // _CHUNK
_EREM = _EPW - _NFULL * _CHUNK
_ACC_SLAB = 624
_ACC_REM = N_NODES - _ACC_SLAB * _NS


_GDN = lax.GatherDimensionNumbers(offset_dims=(), collapsed_slice_dims=(0,),
                                  start_index_map=(0,))


def _lane_allsum(vec, lanes):
    for kk in (8, 4, 2, 1):
        idx = lanes ^ kk
        vec = vec + lax.gather(vec, idx[:, None], _GDN, (1,),
                               mode=lax.GatherScatterMode.PROMISE_IN_BOUNDS)
    return vec


def _combine_body(ei_hbm, q_hbm, kv_hbm, dkv_hbm,
                  out_hbm, acc_shared,
                  i0, i1, eidx,
                  qb0, kvb0, dkvb0,
                  qb1, kvb1, dkvb1,
                  msgb, gsem0, gsem1):
    c = lax.axis_index("c")
    s = lax.axis_index("s")
    idxs = (i0, i1)
    data = ((qb0, kvb0, dkvb0, gsem0),
            (qb1, kvb1, dkvb1, gsem1))

    zero = jnp.zeros((_LANES,), jnp.float32)

    def zrow(i, _):
        for h in range(HIDDEN // _LANES):
            msgb[i, pl.ds(h * _LANES, _LANES)] = zero
        return 0

    lax.fori_loop(0, _CHUNK, zrow, 0)
    for r in range(_ACC_SLAB // _CHUNK):
        pltpu.sync_copy(msgb, acc_shared.at[pl.ds(s * _ACC_SLAB + r * _CHUNK, _CHUNK)])
    rem0 = _ACC_SLAB - (_ACC_SLAB // _CHUNK) * _CHUNK
    if rem0:
        pltpu.sync_copy(msgb.at[pl.ds(0, rem0)],
                        acc_shared.at[pl.ds(s * _ACC_SLAB + (_ACC_SLAB // _CHUNK) * _CHUNK, rem0)])

    @pl.when(s == _NS - 1)
    def _zero_rem():
        pltpu.sync_copy(msgb.at[pl.ds(0, _ACC_REM)],
                        acc_shared.at[pl.ds(_ACC_SLAB * _NS, _ACC_REM)])

    plsc.subcore_barrier()

    base = c * (N_EDGES // _NC) + s * _EPW
    lanes = lax.iota(jnp.int32, _LANES)

    def load_idx(off, b):
        pltpu.sync_copy(ei_hbm.at[pl.ds(off, _CHUNK)], idxs[b].at[0])
        pltpu.sync_copy(ei_hbm.at[pl.ds(N_EDGES + off, _CHUNK)], idxs[b].at[1])

    def gather_copies(off, b, make):
        qb, kvb, dkvb, gsem = data[b]
        is_, id_ = idxs[b].at[0], idxs[b].at[1]
        f = pltpu.make_async_copy if make else pltpu.async_copy
        return [
            f(q_hbm.at[id_], qb, gsem),
            f(kv_hbm.at[is_], kvb, gsem),
            f(dkv_hbm.at[pl.ds(off, _CHUNK)], dkvb, gsem),
        ]

    def compute(b, n_edges):
        qb, kvb, dkvb, _ = data[b]

        himask = jnp.full((_LANES,), -65536, dtype=jnp.int32)

        def unpack2(w):
            lo = lax.bitcast_convert_type(w << 16, jnp.float32)
            hi = lax.bitcast_convert_type(w & himask, jnp.float32)
            return lo, hi

        @plsc.parallel_loop(0, n_edges, 1, unroll=4)
        def edge(e):
            acc = zero
            mvs = []
            for j in range(HIDDEN // _LANES):
                hs = pl.ds(j * _LANES, _LANES)
                dk_j, dv_j = unpack2(dkvb[e, hs])
                k_j, v_j = unpack2(kvb[e, hs])
                acc = acc + qb[e, hs] * k_j * dk_j
                mvs.append(v_j * dv_j)
            tot = _lane_allsum(acc, lanes)
            attn = tot / (1.0 + jnp.exp(-tot))
            for j in range(HIDDEN // _LANES):
                hs = pl.ds(j * _LANES, _LANES)
                msgb[e, hs] = mvs[j] * attn

    load_idx(base, 0)
    gather_copies(base, 0, make=False)
    load_idx(base + _CHUNK, 1)
    gather_copies(base + _CHUNK, 1, make=False)

    def pair(i, _):
        g = i * 2
        for b in range(2):
            t = g + b
            off = base + t * _CHUNK
            for cp in gather_copies(off, b, make=True):
                cp.wait()
            compute(b, _CHUNK)
            pltpu.sync_copy(msgb, acc_shared.at[idxs[b].at[1]], add=True)

            @pl.when(t + 2 < _NFULL)
            def _prefetch():
                off2 = off + 2 * _CHUNK
                load_idx(off2, b)
                gather_copies(off2, b, make=False)

        return 0

    lax.fori_loop(0, _NFULL // 2, pair, 0)

    if _EREM:
        off_e = base + _NFULL * _CHUNK
        pltpu.sync_copy(ei_hbm.at[pl.ds(off_e, _EREM)], eidx.at[0])
        pltpu.sync_copy(ei_hbm.at[pl.ds(N_EDGES + off_e, _EREM)], eidx.at[1])
        ecps = [
            pltpu.async_copy(q_hbm.at[eidx.at[1]], qb0.at[pl.ds(0, _EREM)], gsem0),
            pltpu.async_copy(kv_hbm.at[eidx.at[0]], kvb0.at[pl.ds(0, _EREM)], gsem0),
            pltpu.async_copy(dkv_hbm.at[pl.ds(off_e, _EREM)], dkvb0.at[pl.ds(0, _EREM)], gsem0),
        ]
        for cp in ecps:
            cp.wait()
        compute(0, _EREM)
        pltpu.sync_copy(msgb.at[pl.ds(0, _EREM)], acc_shared.at[eidx.at[1]], add=True)

    plsc.subcore_barrier()

    row0 = s * _ACC_SLAB
    pltpu.sync_copy(acc_shared.at[pl.ds(row0, _ACC_SLAB)],
                    out_hbm.at[c, pl.ds(row0, _ACC_SLAB)])

    @pl.when(s == _NS - 1)
    def _copy_rem():
        pltpu.sync_copy(acc_shared.at[pl.ds(_ACC_SLAB * _NS, _ACC_REM)],
                        out_hbm.at[c, pl.ds(_ACC_SLAB * _NS, _ACC_REM)])


def _edge_combine(q, kv, dkv, ei):
    mesh = plsc.VectorSubcoreMesh(core_axis_name="c", subcore_axis_name="s")
    f = pl.kernel(
        _combine_body,
        out_type=jax.ShapeDtypeStruct((_NC, N_NODES, HIDDEN), jnp.float32),
        mesh=mesh,
        scratch_types=(
            [pltpu.VMEM_SHARED((N_NODES, HIDDEN), jnp.float32)]
            + [pltpu.VMEM((2, _CHUNK), jnp.int32)] * 2
            + [pltpu.VMEM((2, max(_EREM, 8)), jnp.int32)] * 1
            + [pltpu.VMEM((_CHUNK, HIDDEN), jnp.float32),
               pltpu.VMEM((_CHUNK, HIDDEN), jnp.int32),
               pltpu.VMEM((_CHUNK, HIDDEN), jnp.int32)] * 2
            + [pltpu.VMEM((_CHUNK, HIDDEN), jnp.float32)] * 1
            + [pltpu.SemaphoreType.DMA] * 2
        ),
    )
    partials = f(ei, q, kv, dkv)
    return partials[0], partials[1]


def kernel(x, edge_index, edge_weight, edge_attr, ln_g, ln_b, Wq, bq, Wk, bk,
           Wv, bv, Wo, bo, Wdk, bdk, Wdv, bdv):
    q, kv = _qkv(x, Wq.T, bq, Wk.T, bk, Wv.T, bv, ln_g, ln_b)
    dkv = _dkdv(edge_attr, edge_weight, Wdk.T, bdk, Wdv.T, bdv)
    p0, p1 = _edge_combine(q, kv, dkv, edge_index.astype(jnp.int32).reshape(-1))
    return _final(x, p0, p1, Wo.T, bo)


# speedup vs baseline: 1.0630x; 1.0630x over previous
import functools

import jax
import jax.numpy as jnp
from jax import lax
from jax.experimental import pallas as pl
from jax.experimental.pallas import tpu as pltpu
from jax.experimental.pallas import tpu_sc as plsc

HIDDEN = 128
NUM_RBF = 50
N_NODES = 10000
N_EDGES = 320000
CUTOFF_UPPER = 5.0


ROW_BLK = 1000


def _bf16_pair_pack(a, b):
    lo = lax.bitcast_convert_type(a.astype(jnp.bfloat16), jnp.uint16).astype(jnp.uint32)
    hi = lax.bitcast_convert_type(b.astype(jnp.bfloat16), jnp.uint16).astype(jnp.uint32)
    return lax.bitcast_convert_type(lo | (hi << 16), jnp.int32)


def _qkv_body(x_ref, wq_ref, bq_ref, wk_ref, bk_ref, wv_ref, bv_ref,
              ln_g_ref, ln_b_ref, q_ref, kv_ref):
    x = x_ref[...]
    mu = jnp.mean(x, axis=-1, keepdims=True)
    var = jnp.mean((x - mu) ** 2, axis=-1, keepdims=True)
    xn = (x - mu) / jnp.sqrt(var + 1e-5) * ln_g_ref[...] + ln_b_ref[...]
    q_ref[...] = jnp.dot(xn, wq_ref[...], preferred_element_type=jnp.float32) + bq_ref[...]
    k = jnp.dot(xn, wk_ref[...], preferred_element_type=jnp.float32) + bk_ref[...]
    v = jnp.dot(xn, wv_ref[...], preferred_element_type=jnp.float32) + bv_ref[...]
    kv_ref[...] = _bf16_pair_pack(k, v)


def _qkv(x, WqT, bq, WkT, bk, WvT, bv, ln_g, ln_b):
    n = x.shape[0]
    grid = n // ROW_BLK
    row_spec = pl.BlockSpec((ROW_BLK, HIDDEN), lambda i: (i, 0))
    full_spec = pl.BlockSpec((HIDDEN, HIDDEN), lambda i: (0, 0))
    vec_spec = pl.BlockSpec((1, HIDDEN), lambda i: (0, 0))
    out_sd = jax.ShapeDtypeStruct((n, HIDDEN), jnp.float32)
    out_sd_i = jax.ShapeDtypeStruct((n, HIDDEN), jnp.int32)
    return pl.pallas_call(
        _qkv_body,
        grid=(grid,),
        in_specs=[row_spec, full_spec, vec_spec, full_spec, vec_spec,
                  full_spec, vec_spec, vec_spec, vec_spec],
        out_specs=[row_spec, row_spec],
        out_shape=[out_sd, out_sd_i],
    )(x, WqT, bq.reshape(1, -1), WkT, bk.reshape(1, -1), WvT, bv.reshape(1, -1),
      ln_g.reshape(1, -1), ln_b.reshape(1, -1))


EDGE_BLK = 2560


def _silu(x):
    return x * jax.nn.sigmoid(x)


def _dkdv_body(attr_ref, w_ref, wdk_ref, bdk_ref, wdv_ref, bdv_ref, dk_ref):
    attr = attr_ref[...]
    dk = _silu(jnp.dot(attr, wdk_ref[...], preferred_element_type=jnp.float32) + bdk_ref[...])
    dv = _silu(jnp.dot(attr, wdv_ref[...], preferred_element_type=jnp.float32) + bdv_ref[...])
    r = w_ref[...]
    cut = 0.5 * (jnp.cos(r * (jnp.pi / CUTOFF_UPPER)) + 1.0)
    cut = cut * (r < CUTOFF_UPPER).astype(jnp.float32)
    dk_ref[...] = _bf16_pair_pack(dk, dv * cut[:, :, None].reshape(-1, 1))


def _dkdv(edge_attr, edge_weight, WdkT, bdk, WdvT, bdv):
    e = edge_attr.shape[0]
    grid = e // EDGE_BLK
    attr_spec = pl.BlockSpec((EDGE_BLK, NUM_RBF), lambda i: (i, 0))
    w_spec = pl.BlockSpec((1, EDGE_BLK), lambda i: (0, i))
    wm_spec = pl.BlockSpec((NUM_RBF, HIDDEN), lambda i: (0, 0))
    vec_spec = pl.BlockSpec((1, HIDDEN), lambda i: (0, 0))
    out_spec = pl.BlockSpec((EDGE_BLK, HIDDEN), lambda i: (i, 0))
    out_sd = jax.ShapeDtypeStruct((e, HIDDEN), jnp.int32)
    return pl.pallas_call(
        _dkdv_body,
        grid=(grid,),
        in_specs=[attr_spec, w_spec, wm_spec, vec_spec, wm_spec, vec_spec],
        out_specs=out_spec,
        out_shape=out_sd,
    )(edge_attr, edge_weight.reshape(1, -1), WdkT, bdk.reshape(1, -1),
      WdvT, bdv.reshape(1, -1))


def _final_body(x_ref, p0_ref, p1_ref, wo_ref, bo_ref, out_ref):
    s = p0_ref[...] + p1_ref[...]
    out_ref[...] = x_ref[...] + jnp.dot(s, wo_ref[...], preferred_element_type=jnp.float32) + bo_ref[...]


def _final(x, p0, p1, WoT, bo):
    n = x.shape[0]
    grid = n // ROW_BLK
    row_spec = pl.BlockSpec((ROW_BLK, HIDDEN), lambda i: (i, 0))
    full_spec = pl.BlockSpec((HIDDEN, HIDDEN), lambda i: (0, 0))
    vec_spec = pl.BlockSpec((1, HIDDEN), lambda i: (0, 0))
    return pl.pallas_call(
        _final_body,
        grid=(grid,),
        in_specs=[row_spec, row_spec, row_spec, full_spec, vec_spec],
        out_specs=row_spec,
        out_shape=jax.ShapeDtypeStruct((n, HIDDEN), jnp.float32),
    )(x, p0, p1, WoT, bo.reshape(1, -1))


_NC = 2
_NS = 16
_LANES = 16
_NW = _NC * _NS
_EPW = N_EDGES // _NW
_CHUNK = 48
_NFULL = _EPW // _CHUNK
_EREM = _EPW - _NFULL * _CHUNK
_ACC_SLAB = 624
_ACC_REM = N_NODES - _ACC_SLAB * _NS


_GDN = lax.GatherDimensionNumbers(offset_dims=(), collapsed_slice_dims=(0,),
                                  start_index_map=(0,))


def _lane_allsum(vec, lanes):
    for kk in (8, 4, 2, 1):
        idx = lanes ^ kk
        vec = vec + lax.gather(vec, idx[:, None], _GDN, (1,),
                               mode=lax.GatherScatterMode.PROMISE_IN_BOUNDS)
    return vec


def _combine_body(ei_hbm, q_hbm, kv_hbm, dkv_hbm,
                  out_hbm, acc_shared,
                  i0, i1, eidx,
                  qb0, kvb0, dkvb0,
                  qb1, kvb1, dkvb1,
                  msgb, gsem0, gsem1):
    c = lax.axis_index("c")
    s = lax.axis_index("s")
    idxs = (i0, i1)
    data = ((qb0, kvb0, dkvb0, gsem0),
            (qb1, kvb1, dkvb1, gsem1))

    zero = jnp.zeros((_LANES,), jnp.float32)

    def zrow(i, _):
        for h in range(HIDDEN // _LANES):
            msgb[i, pl.ds(h * _LANES, _LANES)] = zero
        return 0

    lax.fori_loop(0, _CHUNK, zrow, 0)
    for r in range(_ACC_SLAB // _CHUNK):
        pltpu.sync_copy(msgb, acc_shared.at[pl.ds(s * _ACC_SLAB + r * _CHUNK, _CHUNK)])
    rem0 = _ACC_SLAB - (_ACC_SLAB // _CHUNK) * _CHUNK
    if rem0:
        pltpu.sync_copy(msgb.at[pl.ds(0, rem0)],
                        acc_shared.at[pl.ds(s * _ACC_SLAB + (_ACC_SLAB // _CHUNK) * _CHUNK, rem0)])

    @pl.when(s == _NS - 1)
    def _zero_rem():
        pltpu.sync_copy(msgb.at[pl.ds(0, _ACC_REM)],
                        acc_shared.at[pl.ds(_ACC_SLAB * _NS, _ACC_REM)])

    plsc.subcore_barrier()

    base = c * (N_EDGES // _NC) + s * _EPW
    lanes = lax.iota(jnp.int32, _LANES)

    def load_idx(off, b):
        pltpu.sync_copy(ei_hbm.at[pl.ds(off, _CHUNK)], idxs[b].at[0])
        pltpu.sync_copy(ei_hbm.at[pl.ds(N_EDGES + off, _CHUNK)], idxs[b].at[1])

    def gather_copies(off, b, make):
        qb, kvb, dkvb, gsem = data[b]
        is_, id_ = idxs[b].at[0], idxs[b].at[1]
        f = pltpu.make_async_copy if make else pltpu.async_copy
        return [
            f(q_hbm.at[id_], qb, gsem),
            f(kv_hbm.at[is_], kvb, gsem),
            f(dkv_hbm.at[pl.ds(off, _CHUNK)], dkvb, gsem),
        ]

    def compute(b, n_edges):
        qb, kvb, dkvb, _ = data[b]

        himask = jnp.full((_LANES,), -65536, dtype=jnp.int32)

        def unpack2(w):
            lo = lax.bitcast_convert_type(w << 16, jnp.float32)
            hi = lax.bitcast_convert_type(w & himask, jnp.float32)
            return lo, hi

        @plsc.parallel_loop(0, n_edges, 1, unroll=2)
        def edge(e):
            acc = zero
            mvs = []
            for j in range(HIDDEN // _LANES):
                hs = pl.ds(j * _LANES, _LANES)
                dk_j, dv_j = unpack2(dkvb[e, hs])
                k_j, v_j = unpack2(kvb[e, hs])
                acc = acc + qb[e, hs] * k_j * dk_j
                mvs.append(v_j * dv_j)
            tot = _lane_allsum(acc, lanes)
            attn = tot / (1.0 + jnp.exp(-tot))
            for j in range(HIDDEN // _LANES):
                hs = pl.ds(j * _LANES, _LANES)
                msgb[e, hs] = mvs[j] * attn

    load_idx(base, 0)
    gather_copies(base, 0, make=False)
    load_idx(base + _CHUNK, 1)
    gather_copies(base + _CHUNK, 1, make=False)

    def pair(i, _):
        g = i * 2
        for b in range(2):
            t = g + b
            off = base + t * _CHUNK
            for cp in gather_copies(off, b, make=True):
                cp.wait()
            compute(b, _CHUNK)
            pltpu.sync_copy(msgb, acc_shared.at[idxs[b].at[1]], add=True)

            @pl.when(t + 2 < _NFULL)
            def _prefetch():
                off2 = off + 2 * _CHUNK
                load_idx(off2, b)
                gather_copies(off2, b, make=False)

        return 0

    lax.fori_loop(0, _NFULL // 2, pair, 0)

    if _EREM:
        off_e = base + _NFULL * _CHUNK
        pltpu.sync_copy(ei_hbm.at[pl.ds(off_e, _EREM)], eidx.at[0])
        pltpu.sync_copy(ei_hbm.at[pl.ds(N_EDGES + off_e, _EREM)], eidx.at[1])
        ecps = [
            pltpu.async_copy(q_hbm.at[eidx.at[1]], qb0.at[pl.ds(0, _EREM)], gsem0),
            pltpu.async_copy(kv_hbm.at[eidx.at[0]], kvb0.at[pl.ds(0, _EREM)], gsem0),
            pltpu.async_copy(dkv_hbm.at[pl.ds(off_e, _EREM)], dkvb0.at[pl.ds(0, _EREM)], gsem0),
        ]
        for cp in ecps:
            cp.wait()
        compute(0, _EREM)
        pltpu.sync_copy(msgb.at[pl.ds(0, _EREM)], acc_shared.at[eidx.at[1]], add=True)

    plsc.subcore_barrier()

    row0 = s * _ACC_SLAB
    pltpu.sync_copy(acc_shared.at[pl.ds(row0, _ACC_SLAB)],
                    out_hbm.at[c, pl.ds(row0, _ACC_SLAB)])

    @pl.when(s == _NS - 1)
    def _copy_rem():
        pltpu.sync_copy(acc_shared.at[pl.ds(_ACC_SLAB * _NS, _ACC_REM)],
                        out_hbm.at[c, pl.ds(_ACC_SLAB * _NS, _ACC_REM)])


def _edge_combine(q, kv, dkv, ei):
    mesh = plsc.VectorSubcoreMesh(core_axis_name="c", subcore_axis_name="s")
    f = pl.kernel(
        _combine_body,
        out_type=jax.ShapeDtypeStruct((_NC, N_NODES, HIDDEN), jnp.float32),
        mesh=mesh,
        scratch_types=(
            [pltpu.VMEM_SHARED((N_NODES, HIDDEN), jnp.float32)]
            + [pltpu.VMEM((2, _CHUNK), jnp.int32)] * 2
            + [pltpu.VMEM((2, max(_EREM, 8)), jnp.int32)] * 1
            + [pltpu.VMEM((_CHUNK, HIDDEN), jnp.float32),
               pltpu.VMEM((_CHUNK, HIDDEN), jnp.int32),
               pltpu.VMEM((_CHUNK, HIDDEN), jnp.int32)] * 2
            + [pltpu.VMEM((_CHUNK, HIDDEN), jnp.float32)] * 1
            + [pltpu.SemaphoreType.DMA] * 2
        ),
    )
    partials = f(ei, q, kv, dkv)
    return partials[0], partials[1]


def kernel(x, edge_index, edge_weight, edge_attr, ln_g, ln_b, Wq, bq, Wk, bk,
           Wv, bv, Wo, bo, Wdk, bdk, Wdv, bdv):
    q, kv = _qkv(x, Wq.T, bq, Wk.T, bk, Wv.T, bv, ln_g, ln_b)
    dkv = _dkdv(edge_attr, edge_weight, Wdk.T, bdk, Wdv.T, bdv)
    p0, p1 = _edge_combine(q, kv, dkv, edge_index.astype(jnp.int32).reshape(-1))
    return _final(x, p0, p1, Wo.T, bo)


# speedup vs baseline: 1.1392x; 1.0716x over previous
import functools

import jax
import jax.numpy as jnp
from jax import lax
from jax.experimental import pallas as pl
from jax.experimental.pallas import tpu as pltpu
from jax.experimental.pallas import tpu_sc as plsc

HIDDEN = 128
NUM_RBF = 50
N_NODES = 10000
N_EDGES = 320000
CUTOFF_UPPER = 5.0


ROW_BLK = 1000


def _bf16_pair_pack(a, b):
    lo = lax.bitcast_convert_type(a.astype(jnp.bfloat16), jnp.uint16).astype(jnp.uint32)
    hi = lax.bitcast_convert_type(b.astype(jnp.bfloat16), jnp.uint16).astype(jnp.uint32)
    return lax.bitcast_convert_type(lo | (hi << 16), jnp.int32)


def _qkv_body(x_ref, wq_ref, bq_ref, wk_ref, bk_ref, wv_ref, bv_ref,
              ln_g_ref, ln_b_ref, q_ref, kv_ref):
    x = x_ref[...]
    mu = jnp.mean(x, axis=-1, keepdims=True)
    var = jnp.mean((x - mu) ** 2, axis=-1, keepdims=True)
    xn = (x - mu) / jnp.sqrt(var + 1e-5) * ln_g_ref[...] + ln_b_ref[...]
    q_ref[...] = jnp.dot(xn, wq_ref[...], preferred_element_type=jnp.float32) + bq_ref[...]
    k = jnp.dot(xn, wk_ref[...], preferred_element_type=jnp.float32) + bk_ref[...]
    v = jnp.dot(xn, wv_ref[...], preferred_element_type=jnp.float32) + bv_ref[...]
    kv_ref[...] = _bf16_pair_pack(k, v)


def _qkv(x, WqT, bq, WkT, bk, WvT, bv, ln_g, ln_b):
    n = x.shape[0]
    grid = n // ROW_BLK
    row_spec = pl.BlockSpec((ROW_BLK, HIDDEN), lambda i: (i, 0))
    full_spec = pl.BlockSpec((HIDDEN, HIDDEN), lambda i: (0, 0))
    vec_spec = pl.BlockSpec((1, HIDDEN), lambda i: (0, 0))
    out_sd = jax.ShapeDtypeStruct((n, HIDDEN), jnp.float32)
    out_sd_i = jax.ShapeDtypeStruct((n, HIDDEN), jnp.int32)
    return pl.pallas_call(
        _qkv_body,
        grid=(grid,),
        in_specs=[row_spec, full_spec, vec_spec, full_spec, vec_spec,
                  full_spec, vec_spec, vec_spec, vec_spec],
        out_specs=[row_spec, row_spec],
        out_shape=[out_sd, out_sd_i],
    )(x, WqT, bq.reshape(1, -1), WkT, bk.reshape(1, -1), WvT, bv.reshape(1, -1),
      ln_g.reshape(1, -1), ln_b.reshape(1, -1))


EDGE_BLK = 2560


def _silu(x):
    return x * jax.nn.sigmoid(x)


def _dkdv_body(attr_ref, w_ref, wdk_ref, bdk_ref, wdv_ref, bdv_ref, dk_ref):
    attr = attr_ref[...]
    dk = _silu(jnp.dot(attr, wdk_ref[...], preferred_element_type=jnp.float32) + bdk_ref[...])
    dv = _silu(jnp.dot(attr, wdv_ref[...], preferred_element_type=jnp.float32) + bdv_ref[...])
    r = w_ref[...]
    cut = 0.5 * (jnp.cos(r * (jnp.pi / CUTOFF_UPPER)) + 1.0)
    cut = cut * (r < CUTOFF_UPPER).astype(jnp.float32)
    dk_ref[...] = _bf16_pair_pack(dk, dv * cut[:, :, None].reshape(-1, 1))


def _dkdv(edge_attr, edge_weight, WdkT, bdk, WdvT, bdv):
    e = edge_attr.shape[0]
    grid = e // EDGE_BLK
    attr_spec = pl.BlockSpec((EDGE_BLK, NUM_RBF), lambda i: (i, 0))
    w_spec = pl.BlockSpec((1, EDGE_BLK), lambda i: (0, i))
    wm_spec = pl.BlockSpec((NUM_RBF, HIDDEN), lambda i: (0, 0))
    vec_spec = pl.BlockSpec((1, HIDDEN), lambda i: (0, 0))
    out_spec = pl.BlockSpec((EDGE_BLK, HIDDEN), lambda i: (i, 0))
    out_sd = jax.ShapeDtypeStruct((e, HIDDEN), jnp.int32)
    return pl.pallas_call(
        _dkdv_body,
        grid=(grid,),
        in_specs=[attr_spec, w_spec, wm_spec, vec_spec, wm_spec, vec_spec],
        out_specs=out_spec,
        out_shape=out_sd,
    )(edge_attr, edge_weight.reshape(1, -1), WdkT, bdk.reshape(1, -1),
      WdvT, bdv.reshape(1, -1))


def _final_body(x_ref, p0_ref, p1_ref, wo_ref, bo_ref, out_ref):
    s = p0_ref[...] + p1_ref[...]
    out_ref[...] = x_ref[...] + jnp.dot(s, wo_ref[...], preferred_element_type=jnp.float32) + bo_ref[...]


def _final(x, p0, p1, WoT, bo):
    n = x.shape[0]
    grid = n // ROW_BLK
    row_spec = pl.BlockSpec((ROW_BLK, HIDDEN), lambda i: (i, 0))
    full_spec = pl.BlockSpec((HIDDEN, HIDDEN), lambda i: (0, 0))
    vec_spec = pl.BlockSpec((1, HIDDEN), lambda i: (0, 0))
    return pl.pallas_call(
        _final_body,
        grid=(grid,),
        in_specs=[row_spec, row_spec, row_spec, full_spec, vec_spec],
        out_specs=row_spec,
        out_shape=jax.ShapeDtypeStruct((n, HIDDEN), jnp.float32),
    )(x, p0, p1, WoT, bo.reshape(1, -1))


_NC = 2
_NS = 16
_LANES = 16
_NW = _NC * _NS
_EPW = N_EDGES // _NW
_CHUNK = 48
_NFULL = _EPW // _CHUNK
_EREM = _EPW - _NFULL * _CHUNK
_ACC_SLAB = 624
_ACC_REM = N_NODES - _ACC_SLAB * _NS


_GDN = lax.GatherDimensionNumbers(offset_dims=(), collapsed_slice_dims=(0,),
                                  start_index_map=(0,))


def _lane_allsum(vec, lanes):
    for kk in (8, 4, 2, 1):
        idx = lanes ^ kk
        vec = vec + lax.gather(vec, idx[:, None], _GDN, (1,),
                               mode=lax.GatherScatterMode.PROMISE_IN_BOUNDS)
    return vec


def _combine_body(ei_hbm, q_hbm, kv_hbm, dkv_hbm,
                  out_hbm, acc_shared,
                  i0s, i0d, i1s, i1d, i2s, i2d, i3s, i3d, eis, eid,
                  qb0, kvb0, dkvb0,
                  qb1, kvb1, dkvb1,
                  mb0, mb1, gsem0, gsem1, ssem0, ssem1):
    c = lax.axis_index("c")
    s = lax.axis_index("s")
    idxs = ((i0s, i0d), (i1s, i1d), (i2s, i2d), (i3s, i3d))
    data = ((qb0, kvb0, dkvb0, gsem0),
            (qb1, kvb1, dkvb1, gsem1))
    msgs = (mb0, mb1)
    ssems = (ssem0, ssem1)
    msgb = mb0

    zero = jnp.zeros((_LANES,), jnp.float32)

    def zrow(i, _):
        for h in range(HIDDEN // _LANES):
            msgb[i, pl.ds(h * _LANES, _LANES)] = zero
        return 0

    lax.fori_loop(0, _CHUNK, zrow, 0)
    for r in range(_ACC_SLAB // _CHUNK):
        pltpu.sync_copy(msgb, acc_shared.at[pl.ds(s * _ACC_SLAB + r * _CHUNK, _CHUNK)])
    rem0 = _ACC_SLAB - (_ACC_SLAB // _CHUNK) * _CHUNK
    if rem0:
        pltpu.sync_copy(msgb.at[pl.ds(0, rem0)],
                        acc_shared.at[pl.ds(s * _ACC_SLAB + (_ACC_SLAB // _CHUNK) * _CHUNK, rem0)])

    @pl.when(s == _NS - 1)
    def _zero_rem():
        pltpu.sync_copy(msgb.at[pl.ds(0, _ACC_REM)],
                        acc_shared.at[pl.ds(_ACC_SLAB * _NS, _ACC_REM)])

    plsc.subcore_barrier()

    base = c * (N_EDGES // _NC) + s * _EPW
    lanes = lax.iota(jnp.int32, _LANES)

    def load_idx(off, sl):
        pltpu.sync_copy(ei_hbm.at[pl.ds(off, _CHUNK)], idxs[sl][0])
        pltpu.sync_copy(ei_hbm.at[pl.ds(N_EDGES + off, _CHUNK)], idxs[sl][1])

    def gather_copies(off, sl, d, make):
        qb, kvb, dkvb, gsem = data[d]
        is_, id_ = idxs[sl]
        f = pltpu.make_async_copy if make else pltpu.async_copy
        return [
            f(q_hbm.at[id_], qb, gsem),
            f(kv_hbm.at[is_], kvb, gsem),
            f(dkv_hbm.at[pl.ds(off, _CHUNK)], dkvb, gsem),
        ]

    def compute(b, n_edges):
        qb, kvb, dkvb, _ = data[b]
        msgb = msgs[b]

        himask = jnp.full((_LANES,), -65536, dtype=jnp.int32)

        def unpack2(w):
            lo = lax.bitcast_convert_type(w << 16, jnp.float32)
            hi = lax.bitcast_convert_type(w & himask, jnp.float32)
            return lo, hi

        @plsc.parallel_loop(0, n_edges, 1, unroll=2)
        def edge(e):
            acc = zero
            mvs = []
            for j in range(HIDDEN // _LANES):
                hs = pl.ds(j * _LANES, _LANES)
                dk_j, dv_j = unpack2(dkvb[e, hs])
                k_j, v_j = unpack2(kvb[e, hs])
                acc = acc + qb[e, hs] * k_j * dk_j
                mvs.append(v_j * dv_j)
            tot = _lane_allsum(acc, lanes)
            attn = tot / (1.0 + jnp.exp(-tot))
            for j in range(HIDDEN // _LANES):
                hs = pl.ds(j * _LANES, _LANES)
                msgb[e, hs] = mvs[j] * attn

    load_idx(base, 0)
    gather_copies(base, 0, 0, make=False)
    load_idx(base + _CHUNK, 1)
    gather_copies(base + _CHUNK, 1, 1, make=False)

    def scatter_copy(sl, d, make):
        f = pltpu.make_async_copy if make else pltpu.async_copy
        if make:
            return pltpu.make_async_copy(msgs[d], acc_shared.at[idxs[sl][1]], ssems[d])
        return pltpu.async_copy(msgs[d], acc_shared.at[idxs[sl][1]], ssems[d], add=True)

    def quad(i, _):
        g = i * 4
        for b in range(4):
            t = g + b
            d = b & 1
            sl2 = (b + 2) & 3
            off = base + t * _CHUNK
            for cp in gather_copies(off, b, d, make=True):
                cp.wait()

            @pl.when(t >= 2)
            def _drain():
                scatter_copy(sl2, d, make=True).wait()

            compute(d, _CHUNK)
            scatter_copy(b, d, make=False)

            @pl.when(t + 2 < _NFULL)
            def _prefetch():
                off2 = off + 2 * _CHUNK
                load_idx(off2, sl2)
                gather_copies(off2, sl2, d, make=False)

        return 0

    lax.fori_loop(0, _NFULL // 4, quad, 0)
    scatter_copy(2, 0, make=True).wait()
    scatter_copy(3, 1, make=True).wait()

    if _EREM:
        off_e = base + _NFULL * _CHUNK
        pltpu.sync_copy(ei_hbm.at[pl.ds(off_e, _EREM)], eis)
        pltpu.sync_copy(ei_hbm.at[pl.ds(N_EDGES + off_e, _EREM)], eid)
        ecps = [
            pltpu.async_copy(q_hbm.at[eid], qb0.at[pl.ds(0, _EREM)], gsem0),
            pltpu.async_copy(kv_hbm.at[eis], kvb0.at[pl.ds(0, _EREM)], gsem0),
            pltpu.async_copy(dkv_hbm.at[pl.ds(off_e, _EREM)], dkvb0.at[pl.ds(0, _EREM)], gsem0),
        ]
        for cp in ecps:
            cp.wait()
        compute(0, _EREM)
        pltpu.sync_copy(mb0.at[pl.ds(0, _EREM)], acc_shared.at[eid], add=True)

    plsc.subcore_barrier()

    row0 = s * _ACC_SLAB
    pltpu.sync_copy(acc_shared.at[pl.ds(row0, _ACC_SLAB)],
                    out_hbm.at[c, pl.ds(row0, _ACC_SLAB)])

    @pl.when(s == _NS - 1)
    def _copy_rem():
        pltpu.sync_copy(acc_shared.at[pl.ds(_ACC_SLAB * _NS, _ACC_REM)],
                        out_hbm.at[c, pl.ds(_ACC_SLAB * _NS, _ACC_REM)])


def _edge_combine(q, kv, dkv, ei):
    mesh = plsc.VectorSubcoreMesh(core_axis_name="c", subcore_axis_name="s")
    f = pl.kernel(
        _combine_body,
        out_type=jax.ShapeDtypeStruct((_NC, N_NODES, HIDDEN), jnp.float32),
        mesh=mesh,
        scratch_types=(
            [pltpu.VMEM_SHARED((N_NODES, HIDDEN), jnp.float32)]
            + [pltpu.VMEM((_CHUNK,), jnp.int32)] * 8
            + [pltpu.VMEM((max(_EREM, 8),), jnp.int32)] * 2
            + [pltpu.VMEM((_CHUNK, HIDDEN), jnp.float32),
               pltpu.VMEM((_CHUNK, HIDDEN), jnp.int32),
               pltpu.VMEM((_CHUNK, HIDDEN), jnp.int32)] * 2
            + [pltpu.VMEM((_CHUNK, HIDDEN), jnp.float32)] * 2
            + [pltpu.SemaphoreType.DMA] * 4
        ),
    )
    partials = f(ei, q, kv, dkv)
    return partials[0], partials[1]


def kernel(x, edge_index, edge_weight, edge_attr, ln_g, ln_b, Wq, bq, Wk, bk,
           Wv, bv, Wo, bo, Wdk, bdk, Wdv, bdv):
    q, kv = _qkv(x, Wq.T, bq, Wk.T, bk, Wv.T, bv, ln_g, ln_b)
    dkv = _dkdv(edge_attr, edge_weight, Wdk.T, bdk, Wdv.T, bdv)
    p0, p1 = _edge_combine(q, kv, dkv, edge_index.astype(jnp.int32).reshape(-1))
    return _final(x, p0, p1, Wo.T, bo)


# speedup vs baseline: 1.2147x; 1.0663x over previous
import functools

import jax
import jax.numpy as jnp
from jax import lax
from jax.experimental import pallas as pl
from jax.experimental.pallas import tpu as pltpu
from jax.experimental.pallas import tpu_sc as plsc

HIDDEN = 128
NUM_RBF = 50
N_NODES = 10000
N_EDGES = 320000
CUTOFF_UPPER = 5.0


ROW_BLK = 1000


def _bf16_pair_pack(a, b):
    lo = lax.bitcast_convert_type(a.astype(jnp.bfloat16), jnp.uint16).astype(jnp.uint32)
    hi = lax.bitcast_convert_type(b.astype(jnp.bfloat16), jnp.uint16).astype(jnp.uint32)
    return lax.bitcast_convert_type(lo | (hi << 16), jnp.int32)


def _qkv_body(x_ref, wq_ref, bq_ref, wk_ref, bk_ref, wv_ref, bv_ref,
              ln_g_ref, ln_b_ref, q_ref, kv_ref):
    x = x_ref[...]
    mu = jnp.mean(x, axis=-1, keepdims=True)
    var = jnp.mean((x - mu) ** 2, axis=-1, keepdims=True)
    xn = (x - mu) / jnp.sqrt(var + 1e-5) * ln_g_ref[...] + ln_b_ref[...]
    q_ref[...] = jnp.dot(xn, wq_ref[...], preferred_element_type=jnp.float32) + bq_ref[...]
    k = jnp.dot(xn, wk_ref[...], preferred_element_type=jnp.float32) + bk_ref[...]
    v = jnp.dot(xn, wv_ref[...], preferred_element_type=jnp.float32) + bv_ref[...]
    kv_ref[...] = _bf16_pair_pack(k, v)


def _qkv(x, WqT, bq, WkT, bk, WvT, bv, ln_g, ln_b):
    n = x.shape[0]
    grid = n // ROW_BLK
    row_spec = pl.BlockSpec((ROW_BLK, HIDDEN), lambda i: (i, 0))
    full_spec = pl.BlockSpec((HIDDEN, HIDDEN), lambda i: (0, 0))
    vec_spec = pl.BlockSpec((1, HIDDEN), lambda i: (0, 0))
    out_sd = jax.ShapeDtypeStruct((n, HIDDEN), jnp.float32)
    out_sd_i = jax.ShapeDtypeStruct((n, HIDDEN), jnp.int32)
    return pl.pallas_call(
        _qkv_body,
        grid=(grid,),
        in_specs=[row_spec, full_spec, vec_spec, full_spec, vec_spec,
                  full_spec, vec_spec, vec_spec, vec_spec],
        out_specs=[row_spec, row_spec],
        out_shape=[out_sd, out_sd_i],
    )(x, WqT, bq.reshape(1, -1), WkT, bk.reshape(1, -1), WvT, bv.reshape(1, -1),
      ln_g.reshape(1, -1), ln_b.reshape(1, -1))


EDGE_BLK = 6400


def _silu(x):
    return x * jax.nn.sigmoid(x)


def _dkdv_body(attr_ref, w_ref, wdk_ref, bdk_ref, wdv_ref, bdv_ref, dk_ref):
    attr = attr_ref[...]
    dk = _silu(jnp.dot(attr, wdk_ref[...], preferred_element_type=jnp.float32) + bdk_ref[...])
    dv = _silu(jnp.dot(attr, wdv_ref[...], preferred_element_type=jnp.float32) + bdv_ref[...])
    r = w_ref[...]
    cut = 0.5 * (jnp.cos(r * (jnp.pi / CUTOFF_UPPER)) + 1.0)
    cut = cut * (r < CUTOFF_UPPER).astype(jnp.float32)
    dk_ref[...] = _bf16_pair_pack(dk, dv * cut[:, :, None].reshape(-1, 1))


def _dkdv(edge_attr, edge_weight, WdkT, bdk, WdvT, bdv):
    e = edge_attr.shape[0]
    grid = e // EDGE_BLK
    attr_spec = pl.BlockSpec((EDGE_BLK, NUM_RBF), lambda i: (i, 0))
    w_spec = pl.BlockSpec((1, EDGE_BLK), lambda i: (0, i))
    wm_spec = pl.BlockSpec((NUM_RBF, HIDDEN), lambda i: (0, 0))
    vec_spec = pl.BlockSpec((1, HIDDEN), lambda i: (0, 0))
    out_spec = pl.BlockSpec((EDGE_BLK, HIDDEN), lambda i: (i, 0))
    out_sd = jax.ShapeDtypeStruct((e, HIDDEN), jnp.int32)
    return pl.pallas_call(
        _dkdv_body,
        grid=(grid,),
        in_specs=[attr_spec, w_spec, wm_spec, vec_spec, wm_spec, vec_spec],
        out_specs=out_spec,
        out_shape=out_sd,
    )(edge_attr, edge_weight.reshape(1, -1), WdkT, bdk.reshape(1, -1),
      WdvT, bdv.reshape(1, -1))


def _final_body(x_ref, p0_ref, p1_ref, wo_ref, bo_ref, out_ref):
    s = p0_ref[...] + p1_ref[...]
    out_ref[...] = x_ref[...] + jnp.dot(s, wo_ref[...], preferred_element_type=jnp.float32) + bo_ref[...]


def _final(x, p0, p1, WoT, bo):
    n = x.shape[0]
    grid = n // ROW_BLK
    row_spec = pl.BlockSpec((ROW_BLK, HIDDEN), lambda i: (i, 0))
    full_spec = pl.BlockSpec((HIDDEN, HIDDEN), lambda i: (0, 0))
    vec_spec = pl.BlockSpec((1, HIDDEN), lambda i: (0, 0))
    return pl.pallas_call(
        _final_body,
        grid=(grid,),
        in_specs=[row_spec, row_spec, row_spec, full_spec, vec_spec],
        out_specs=row_spec,
        out_shape=jax.ShapeDtypeStruct((n, HIDDEN), jnp.float32),
    )(x, p0, p1, WoT, bo.reshape(1, -1))


_NC = 2
_NS = 16
_LANES = 16
_NW = _NC * _NS
_EPW = N_EDGES // _NW
_CHUNK = 48
_NFULL = _EPW // _CHUNK
_EREM = _EPW - _NFULL * _CHUNK
_ACC_SLAB = 624
_ACC_REM = N_NODES - _ACC_SLAB * _NS


_GDN = lax.GatherDimensionNumbers(offset_dims=(), collapsed_slice_dims=(0,),
                                  start_index_map=(0,))


def _lane_allsum(vec, lanes):
    for kk in (8, 4, 2, 1):
        idx = lanes ^ kk
        vec = vec + lax.gather(vec, idx[:, None], _GDN, (1,),
                               mode=lax.GatherScatterMode.PROMISE_IN_BOUNDS)
    return vec


def _combine_body(ei_hbm, q_hbm, kv_hbm, dkv_hbm,
                  out_hbm, acc_shared,
                  i0s, i0d, i1s, i1d, i2s, i2d, i3s, i3d, eis, eid,
                  qb0, kvb0, dkvb0,
                  qb1, kvb1, dkvb1,
                  mb0, mb1, gsem0, gsem1, ssem0, ssem1):
    c = lax.axis_index("c")
    s = lax.axis_index("s")
    idxs = ((i0s, i0d), (i1s, i1d), (i2s, i2d), (i3s, i3d))
    data = ((qb0, kvb0, dkvb0, gsem0),
            (qb1, kvb1, dkvb1, gsem1))
    msgs = (mb0, mb1)
    ssems = (ssem0, ssem1)
    msgb = mb0

    zero = jnp.zeros((_LANES,), jnp.float32)

    def zrow(i, _):
        for h in range(HIDDEN // _LANES):
            msgb[i, pl.ds(h * _LANES, _LANES)] = zero
        return 0

    lax.fori_loop(0, _CHUNK, zrow, 0)
    for r in range(_ACC_SLAB // _CHUNK):
        pltpu.sync_copy(msgb, acc_shared.at[pl.ds(s * _ACC_SLAB + r * _CHUNK, _CHUNK)])
    rem0 = _ACC_SLAB - (_ACC_SLAB // _CHUNK) * _CHUNK
    if rem0:
        pltpu.sync_copy(msgb.at[pl.ds(0, rem0)],
                        acc_shared.at[pl.ds(s * _ACC_SLAB + (_ACC_SLAB // _CHUNK) * _CHUNK, rem0)])

    @pl.when(s == _NS - 1)
    def _zero_rem():
        pltpu.sync_copy(msgb.at[pl.ds(0, _ACC_REM)],
                        acc_shared.at[pl.ds(_ACC_SLAB * _NS, _ACC_REM)])

    plsc.subcore_barrier()

    base = c * (N_EDGES // _NC) + s * _EPW
    lanes = lax.iota(jnp.int32, _LANES)

    def load_idx(off, sl):
        pltpu.sync_copy(ei_hbm.at[pl.ds(off, _CHUNK)], idxs[sl][0])
        pltpu.sync_copy(ei_hbm.at[pl.ds(N_EDGES + off, _CHUNK)], idxs[sl][1])

    def gather_copies(off, sl, d, make):
        qb, kvb, dkvb, gsem = data[d]
        is_, id_ = idxs[sl]
        f = pltpu.make_async_copy if make else pltpu.async_copy
        return [
            f(q_hbm.at[id_], qb, gsem),
            f(kv_hbm.at[is_], kvb, gsem),
            f(dkv_hbm.at[pl.ds(off, _CHUNK)], dkvb, gsem),
        ]

    def compute(b, n_edges):
        qb, kvb, dkvb, _ = data[b]
        msgb = msgs[b]

        himask = jnp.full((_LANES,), -65536, dtype=jnp.int32)

        def unpack2(w):
            lo = lax.bitcast_convert_type(w << 16, jnp.float32)
            hi = lax.bitcast_convert_type(w & himask, jnp.float32)
            return lo, hi

        @plsc.parallel_loop(0, n_edges, 1, unroll=2)
        def edge(e):
            acc = zero
            mvs = []
            for j in range(HIDDEN // _LANES):
                hs = pl.ds(j * _LANES, _LANES)
                dk_j, dv_j = unpack2(dkvb[e, hs])
                k_j, v_j = unpack2(kvb[e, hs])
                acc = acc + qb[e, hs] * k_j * dk_j
                mvs.append(v_j * dv_j)
            tot = _lane_allsum(acc, lanes)
            attn = tot / (1.0 + jnp.exp(-tot))
            for j in range(HIDDEN // _LANES):
                hs = pl.ds(j * _LANES, _LANES)
                msgb[e, hs] = mvs[j] * attn

    load_idx(base, 0)
    gather_copies(base, 0, 0, make=False)
    load_idx(base + _CHUNK, 1)
    gather_copies(base + _CHUNK, 1, 1, make=False)

    def scatter_copy(sl, d, make):
        f = pltpu.make_async_copy if make else pltpu.async_copy
        if make:
            return pltpu.make_async_copy(msgs[d], acc_shared.at[idxs[sl][1]], ssems[d])
        return pltpu.async_copy(msgs[d], acc_shared.at[idxs[sl][1]], ssems[d], add=True)

    def quad(i, _):
        g = i * 4
        for b in range(4):
            t = g + b
            d = b & 1
            sl2 = (b + 2) & 3
            off = base + t * _CHUNK
            for cp in gather_copies(off, b, d, make=True):
                cp.wait()

            @pl.when(t >= 2)
            def _drain():
                scatter_copy(sl2, d, make=True).wait()

            compute(d, _CHUNK)
            scatter_copy(b, d, make=False)

            @pl.when(t + 2 < _NFULL)
            def _prefetch():
                off2 = off + 2 * _CHUNK
                load_idx(off2, sl2)
                gather_copies(off2, sl2, d, make=False)

        return 0

    lax.fori_loop(0, _NFULL // 4, quad, 0)
    scatter_copy(2, 0, make=True).wait()
    scatter_copy(3, 1, make=True).wait()

    if _EREM:
        off_e = base + _NFULL * _CHUNK
        pltpu.sync_copy(ei_hbm.at[pl.ds(off_e, _EREM)], eis)
        pltpu.sync_copy(ei_hbm.at[pl.ds(N_EDGES + off_e, _EREM)], eid)
        ecps = [
            pltpu.async_copy(q_hbm.at[eid], qb0.at[pl.ds(0, _EREM)], gsem0),
            pltpu.async_copy(kv_hbm.at[eis], kvb0.at[pl.ds(0, _EREM)], gsem0),
            pltpu.async_copy(dkv_hbm.at[pl.ds(off_e, _EREM)], dkvb0.at[pl.ds(0, _EREM)], gsem0),
        ]
        for cp in ecps:
            cp.wait()
        compute(0, _EREM)
        pltpu.sync_copy(mb0.at[pl.ds(0, _EREM)], acc_shared.at[eid], add=True)

    plsc.subcore_barrier()

    row0 = s * _ACC_SLAB
    pltpu.sync_copy(acc_shared.at[pl.ds(row0, _ACC_SLAB)],
                    out_hbm.at[c, pl.ds(row0, _ACC_SLAB)])

    @pl.when(s == _NS - 1)
    def _copy_rem():
        pltpu.sync_copy(acc_shared.at[pl.ds(_ACC_SLAB * _NS, _ACC_REM)],
                        out_hbm.at[c, pl.ds(_ACC_SLAB * _NS, _ACC_REM)])


def _edge_combine(q, kv, dkv, ei):
    mesh = plsc.VectorSubcoreMesh(core_axis_name="c", subcore_axis_name="s")
    f = pl.kernel(
        _combine_body,
        out_type=jax.ShapeDtypeStruct((_NC, N_NODES, HIDDEN), jnp.float32),
        mesh=mesh,
        scratch_types=(
            [pltpu.VMEM_SHARED((N_NODES, HIDDEN), jnp.float32)]
            + [pltpu.VMEM((_CHUNK,), jnp.int32)] * 8
            + [pltpu.VMEM((max(_EREM, 8),), jnp.int32)] * 2
            + [pltpu.VMEM((_CHUNK, HIDDEN), jnp.float32),
               pltpu.VMEM((_CHUNK, HIDDEN), jnp.int32),
               pltpu.VMEM((_CHUNK, HIDDEN), jnp.int32)] * 2
            + [pltpu.VMEM((_CHUNK, HIDDEN), jnp.float32)] * 2
            + [pltpu.SemaphoreType.DMA] * 4
        ),
    )
    partials = f(ei, q, kv, dkv)
    return partials[0], partials[1]


def kernel(x, edge_index, edge_weight, edge_attr, ln_g, ln_b, Wq, bq, Wk, bk,
           Wv, bv, Wo, bo, Wdk, bdk, Wdv, bdv):
    q, kv = _qkv(x, Wq.T, bq, Wk.T, bk, Wv.T, bv, ln_g, ln_b)
    dkv = _dkdv(edge_attr, edge_weight, Wdk.T, bdk, Wdv.T, bdv)
    p0, p1 = _edge_combine(q, kv, dkv, edge_index.astype(jnp.int32).reshape(-1))
    return _final(x, p0, p1, Wo.T, bo)


# speedup vs baseline: 1.2504x; 1.0294x over previous
import functools

import jax
import jax.numpy as jnp
from jax import lax
from jax.experimental import pallas as pl
from jax.experimental.pallas import tpu as pltpu
from jax.experimental.pallas import tpu_sc as plsc

HIDDEN = 128
NUM_RBF = 50
N_NODES = 10000
N_EDGES = 320000
CUTOFF_UPPER = 5.0


ROW_BLK = 2000


def _bf16_pair_pack(a, b):
    lo = lax.bitcast_convert_type(a.astype(jnp.bfloat16), jnp.uint16).astype(jnp.uint32)
    hi = lax.bitcast_convert_type(b.astype(jnp.bfloat16), jnp.uint16).astype(jnp.uint32)
    return lax.bitcast_convert_type(lo | (hi << 16), jnp.int32)


def _qkv_body(x_ref, wq_ref, bq_ref, wk_ref, bk_ref, wv_ref, bv_ref,
              ln_g_ref, ln_b_ref, q_ref, kv_ref):
    x = x_ref[...]
    mu = jnp.mean(x, axis=-1, keepdims=True)
    var = jnp.mean((x - mu) ** 2, axis=-1, keepdims=True)
    xn = (x - mu) / jnp.sqrt(var + 1e-5) * ln_g_ref[...] + ln_b_ref[...]
    q_ref[...] = jnp.dot(xn, wq_ref[...], preferred_element_type=jnp.float32) + bq_ref[...]
    k = jnp.dot(xn, wk_ref[...], preferred_element_type=jnp.float32) + bk_ref[...]
    v = jnp.dot(xn, wv_ref[...], preferred_element_type=jnp.float32) + bv_ref[...]
    kv_ref[...] = _bf16_pair_pack(k, v)


def _qkv(x, WqT, bq, WkT, bk, WvT, bv, ln_g, ln_b):
    n = x.shape[0]
    grid = n // ROW_BLK
    row_spec = pl.BlockSpec((ROW_BLK, HIDDEN), lambda i: (i, 0))
    full_spec = pl.BlockSpec((HIDDEN, HIDDEN), lambda i: (0, 0))
    vec_spec = pl.BlockSpec((1, HIDDEN), lambda i: (0, 0))
    out_sd = jax.ShapeDtypeStruct((n, HIDDEN), jnp.float32)
    out_sd_i = jax.ShapeDtypeStruct((n, HIDDEN), jnp.int32)
    return pl.pallas_call(
        _qkv_body,
        grid=(grid,),
        in_specs=[row_spec, full_spec, vec_spec, full_spec, vec_spec,
                  full_spec, vec_spec, vec_spec, vec_spec],
        out_specs=[row_spec, row_spec],
        out_shape=[out_sd, out_sd_i],
    )(x, WqT, bq.reshape(1, -1), WkT, bk.reshape(1, -1), WvT, bv.reshape(1, -1),
      ln_g.reshape(1, -1), ln_b.reshape(1, -1))


EDGE_BLK = 12800


def _silu(x):
    return x * jax.nn.sigmoid(x)


def _dkdv_body(attr_ref, w_ref, wdk_ref, bdk_ref, wdv_ref, bdv_ref, dk_ref):
    attr = attr_ref[...]
    dk = _silu(jnp.dot(attr, wdk_ref[...], preferred_element_type=jnp.float32) + bdk_ref[...])
    dv = _silu(jnp.dot(attr, wdv_ref[...], preferred_element_type=jnp.float32) + bdv_ref[...])
    r = w_ref[...]
    cut = 0.5 * (jnp.cos(r * (jnp.pi / CUTOFF_UPPER)) + 1.0)
    cut = cut * (r < CUTOFF_UPPER).astype(jnp.float32)
    dk_ref[...] = _bf16_pair_pack(dk, dv * cut[:, :, None].reshape(-1, 1))


def _dkdv(edge_attr, edge_weight, WdkT, bdk, WdvT, bdv):
    e = edge_attr.shape[0]
    grid = e // EDGE_BLK
    attr_spec = pl.BlockSpec((EDGE_BLK, NUM_RBF), lambda i: (i, 0))
    w_spec = pl.BlockSpec((1, EDGE_BLK), lambda i: (0, i))
    wm_spec = pl.BlockSpec((NUM_RBF, HIDDEN), lambda i: (0, 0))
    vec_spec = pl.BlockSpec((1, HIDDEN), lambda i: (0, 0))
    out_spec = pl.BlockSpec((EDGE_BLK, HIDDEN), lambda i: (i, 0))
    out_sd = jax.ShapeDtypeStruct((e, HIDDEN), jnp.int32)
    return pl.pallas_call(
        _dkdv_body,
        grid=(grid,),
        in_specs=[attr_spec, w_spec, wm_spec, vec_spec, wm_spec, vec_spec],
        out_specs=out_spec,
        out_shape=out_sd,
    )(edge_attr, edge_weight.reshape(1, -1), WdkT, bdk.reshape(1, -1),
      WdvT, bdv.reshape(1, -1))


def _final_body(x_ref, p0_ref, p1_ref, wo_ref, bo_ref, out_ref):
    s = p0_ref[...] + p1_ref[...]
    out_ref[...] = x_ref[...] + jnp.dot(s, wo_ref[...], preferred_element_type=jnp.float32) + bo_ref[...]


def _final(x, p0, p1, WoT, bo):
    n = x.shape[0]
    grid = n // ROW_BLK
    row_spec = pl.BlockSpec((ROW_BLK, HIDDEN), lambda i: (i, 0))
    full_spec = pl.BlockSpec((HIDDEN, HIDDEN), lambda i: (0, 0))
    vec_spec = pl.BlockSpec((1, HIDDEN), lambda i: (0, 0))
    return pl.pallas_call(
        _final_body,
        grid=(grid,),
        in_specs=[row_spec, row_spec, row_spec, full_spec, vec_spec],
        out_specs=row_spec,
        out_shape=jax.ShapeDtypeStruct((n, HIDDEN), jnp.float32),
    )(x, p0, p1, WoT, bo.reshape(1, -1))


_NC = 2
_NS = 16
_LANES = 16
_NW = _NC * _NS
_EPW = N_EDGES // _NW
_CHUNK = 48
_NFULL = _EPW // _CHUNK
_EREM = _EPW - _NFULL * _CHUNK
_ACC_SLAB = 624
_ACC_REM = N_NODES - _ACC_SLAB * _NS


_GDN = lax.GatherDimensionNumbers(offset_dims=(), collapsed_slice_dims=(0,),
                                  start_index_map=(0,))


def _lane_allsum(vec, lanes):
    for kk in (8, 4, 2, 1):
        idx = lanes ^ kk
        vec = vec + lax.gather(vec, idx[:, None], _GDN, (1,),
                               mode=lax.GatherScatterMode.PROMISE_IN_BOUNDS)
    return vec


def _combine_body(ei_hbm, q_hbm, kv_hbm, dkv_hbm,
                  out_hbm, acc_shared,
                  i0s, i0d, i1s, i1d, i2s, i2d, i3s, i3d, eis, eid,
                  qb0, kvb0, dkvb0,
                  qb1, kvb1, dkvb1,
                  mb0, mb1, gsem0, gsem1, ssem0, ssem1):
    c = lax.axis_index("c")
    s = lax.axis_index("s")
    idxs = ((i0s, i0d), (i1s, i1d), (i2s, i2d), (i3s, i3d))
    data = ((qb0, kvb0, dkvb0, gsem0),
            (qb1, kvb1, dkvb1, gsem1))
    msgs = (mb0, mb1)
    ssems = (ssem0, ssem1)
    msgb = mb0

    zero = jnp.zeros((_LANES,), jnp.float32)

    def zrow(i, _):
        for h in range(HIDDEN // _LANES):
            msgb[i, pl.ds(h * _LANES, _LANES)] = zero
        return 0

    lax.fori_loop(0, _CHUNK, zrow, 0)
    for r in range(_ACC_SLAB // _CHUNK):
        pltpu.sync_copy(msgb, acc_shared.at[pl.ds(s * _ACC_SLAB + r * _CHUNK, _CHUNK)])
    rem0 = _ACC_SLAB - (_ACC_SLAB // _CHUNK) * _CHUNK
    if rem0:
        pltpu.sync_copy(msgb.at[pl.ds(0, rem0)],
                        acc_shared.at[pl.ds(s * _ACC_SLAB + (_ACC_SLAB // _CHUNK) * _CHUNK, rem0)])

    @pl.when(s == _NS - 1)
    def _zero_rem():
        pltpu.sync_copy(msgb.at[pl.ds(0, _ACC_REM)],
                        acc_shared.at[pl.ds(_ACC_SLAB * _NS, _ACC_REM)])

    plsc.subcore_barrier()

    base = c * (N_EDGES // _NC) + s * _EPW
    lanes = lax.iota(jnp.int32, _LANES)

    def load_idx(off, sl):
        pltpu.sync_copy(ei_hbm.at[pl.ds(off, _CHUNK)], idxs[sl][0])
        pltpu.sync_copy(ei_hbm.at[pl.ds(N_EDGES + off, _CHUNK)], idxs[sl][1])

    def gather_copies(off, sl, d, make):
        qb, kvb, dkvb, gsem = data[d]
        is_, id_ = idxs[sl]
        f = pltpu.make_async_copy if make else pltpu.async_copy
        return [
            f(q_hbm.at[id_], qb, gsem),
            f(kv_hbm.at[is_], kvb, gsem),
            f(dkv_hbm.at[pl.ds(off, _CHUNK)], dkvb, gsem),
        ]

    def compute(b, n_edges):
        qb, kvb, dkvb, _ = data[b]
        msgb = msgs[b]

        himask = jnp.full((_LANES,), -65536, dtype=jnp.int32)

        def unpack2(w):
            lo = lax.bitcast_convert_type(w << 16, jnp.float32)
            hi = lax.bitcast_convert_type(w & himask, jnp.float32)
            return lo, hi

        @plsc.parallel_loop(0, n_edges, 1, unroll=2)
        def edge(e):
            acc = zero
            mvs = []
            for j in range(HIDDEN // _LANES):
                hs = pl.ds(j * _LANES, _LANES)
                dk_j, dv_j = unpack2(dkvb[e, hs])
                k_j, v_j = unpack2(kvb[e, hs])
                acc = acc + qb[e, hs] * k_j * dk_j
                mvs.append(v_j * dv_j)
            tot = _lane_allsum(acc, lanes)
            attn = tot / (1.0 + jnp.exp(-tot))
            for j in range(HIDDEN // _LANES):
                hs = pl.ds(j * _LANES, _LANES)
                msgb[e, hs] = mvs[j] * attn

    load_idx(base, 0)
    gather_copies(base, 0, 0, make=False)
    load_idx(base + _CHUNK, 1)
    gather_copies(base + _CHUNK, 1, 1, make=False)

    def scatter_copy(sl, d, make):
        f = pltpu.make_async_copy if make else pltpu.async_copy
        if make:
            return pltpu.make_async_copy(msgs[d], acc_shared.at[idxs[sl][1]], ssems[d])
        return pltpu.async_copy(msgs[d], acc_shared.at[idxs[sl][1]], ssems[d], add=True)

    def quad(i, _):
        g = i * 4
        for b in range(4):
            t = g + b
            d = b & 1
            sl2 = (b + 2) & 3
            off = base + t * _CHUNK
            for cp in gather_copies(off, b, d, make=True):
                cp.wait()

            @pl.when(t >= 2)
            def _drain():
                scatter_copy(sl2, d, make=True).wait()

            compute(d, _CHUNK)
            scatter_copy(b, d, make=False)

            @pl.when(t + 2 < _NFULL)
            def _prefetch():
                off2 = off + 2 * _CHUNK
                load_idx(off2, sl2)
                gather_copies(off2, sl2, d, make=False)

        return 0

    lax.fori_loop(0, _NFULL // 4, quad, 0)
    scatter_copy(2, 0, make=True).wait()
    scatter_copy(3, 1, make=True).wait()

    if _EREM:
        off_e = base + _NFULL * _CHUNK
        pltpu.sync_copy(ei_hbm.at[pl.ds(off_e, _EREM)], eis)
        pltpu.sync_copy(ei_hbm.at[pl.ds(N_EDGES + off_e, _EREM)], eid)
        ecps = [
            pltpu.async_copy(q_hbm.at[eid], qb0.at[pl.ds(0, _EREM)], gsem0),
            pltpu.async_copy(kv_hbm.at[eis], kvb0.at[pl.ds(0, _EREM)], gsem0),
            pltpu.async_copy(dkv_hbm.at[pl.ds(off_e, _EREM)], dkvb0.at[pl.ds(0, _EREM)], gsem0),
        ]
        for cp in ecps:
            cp.wait()
        compute(0, _EREM)
        pltpu.sync_copy(mb0.at[pl.ds(0, _EREM)], acc_shared.at[eid], add=True)

    plsc.subcore_barrier()

    row0 = s * _ACC_SLAB
    pltpu.sync_copy(acc_shared.at[pl.ds(row0, _ACC_SLAB)],
                    out_hbm.at[c, pl.ds(row0, _ACC_SLAB)])

    @pl.when(s == _NS - 1)
    def _copy_rem():
        pltpu.sync_copy(acc_shared.at[pl.ds(_ACC_SLAB * _NS, _ACC_REM)],
                        out_hbm.at[c, pl.ds(_ACC_SLAB * _NS, _ACC_REM)])


def _edge_combine(q, kv, dkv, ei):
    mesh = plsc.VectorSubcoreMesh(core_axis_name="c", subcore_axis_name="s")
    f = pl.kernel(
        _combine_body,
        out_type=jax.ShapeDtypeStruct((_NC, N_NODES, HIDDEN), jnp.float32),
        mesh=mesh,
        scratch_types=(
            [pltpu.VMEM_SHARED((N_NODES, HIDDEN), jnp.float32)]
            + [pltpu.VMEM((_CHUNK,), jnp.int32)] * 8
            + [pltpu.VMEM((max(_EREM, 8),), jnp.int32)] * 2
            + [pltpu.VMEM((_CHUNK, HIDDEN), jnp.float32),
               pltpu.VMEM((_CHUNK, HIDDEN), jnp.int32),
               pltpu.VMEM((_CHUNK, HIDDEN), jnp.int32)] * 2
            + [pltpu.VMEM((_CHUNK, HIDDEN), jnp.float32)] * 2
            + [pltpu.SemaphoreType.DMA] * 4
        ),
    )
    partials = f(ei, q, kv, dkv)
    return partials[0], partials[1]


def kernel(x, edge_index, edge_weight, edge_attr, ln_g, ln_b, Wq, bq, Wk, bk,
           Wv, bv, Wo, bo, Wdk, bdk, Wdv, bdv):
    q, kv = _qkv(x, Wq.T, bq, Wk.T, bk, Wv.T, bv, ln_g, ln_b)
    dkv = _dkdv(edge_attr, edge_weight, Wdk.T, bdk, Wdv.T, bdv)
    p0, p1 = _edge_combine(q, kv, dkv, edge_index.astype(jnp.int32).reshape(-1))
    return _final(x, p0, p1, Wo.T, bo)
